# baseline (speedup 1.0000x reference)
import functools

import jax
import jax.numpy as jnp
from jax import lax
from jax.experimental import pallas as pl
from jax.experimental.pallas import tpu as pltpu
from jax.experimental.pallas import tpu_sc as plsc

_BATCH = 16384
_TABLE = 1001


def _make_sc_gather():
    info = plsc.get_sparse_core_info()
    ns, lanes = info.num_subcores, info.num_lanes
    b_per_w = _BATCH // ns
    half = b_per_w // 2

    mesh = plsc.VectorSubcoreMesh(
        core_axis_name="c", subcore_axis_name="s", num_cores=1
    )

    @functools.partial(
        pl.kernel,
        mesh=mesh,
        out_type=jax.ShapeDtypeStruct((_BATCH,), jnp.float32),
        scratch_types=[
            pltpu.VMEM((_TABLE,), jnp.float32),
            pltpu.VMEM((b_per_w,), jnp.int32),
            pltpu.VMEM((b_per_w,), jnp.float32),
            pltpu.SemaphoreType.DMA,
            pltpu.SemaphoreType.DMA,
            pltpu.SemaphoreType.DMA,
        ],
        compiler_params=pltpu.CompilerParams(
            needs_layout_passes=False,
            disable_bounds_checks=True,
            disable_semaphore_checks=True,
        ),
    )
    def sc_gather(idx_hbm, betas_hbm, out_hbm, table_v, idx_v, out_v, s1, s2, s3):
        base = lax.axis_index("s") * b_per_w
        cp_tab = pltpu.async_copy(betas_hbm, table_v, s1)
        cp_idx = pltpu.async_copy(idx_hbm.at[pl.ds(base, b_per_w)], idx_v, s2)
        cp_tab.wait()
        cp_idx.wait()

        def body(i, carry):
            iv = idx_v[pl.ds(i * lanes, lanes)]
            out_v[pl.ds(i * lanes, lanes)] = plsc.load_gather(table_v, [iv])
            return carry

        lax.fori_loop(0, half // lanes, body, 0)
        cp_lo = pltpu.async_copy(
            out_v.at[pl.ds(0, half)], out_hbm.at[pl.ds(base, half)], s3
        )
        lax.fori_loop(half // lanes, b_per_w // lanes, body, 0)
        cp_hi = pltpu.async_copy(
            out_v.at[pl.ds(half, half)], out_hbm.at[pl.ds(base + half, half)], s3
        )
        cp_lo.wait()
        cp_hi.wait()

    return sc_gather


_sc_gather = _make_sc_gather()


def kernel(t_int, betas):
    return _sc_gather(t_int.astype(jnp.int32), betas)
